# baseline (speedup 1.0000x reference)
import functools

import jax
import jax.numpy as jnp
from jax import lax
from jax.experimental import pallas as pl
from jax.experimental.pallas import tpu as pltpu
from jax.experimental.pallas import tpu_sc as plsc

_N = 4194304
_PHASE_BINS = 64
_MAG_BINS = 1024
_TWO_PI_OVER_P = 2.0 * 3.141592653589793 / _PHASE_BINS
_INV_MM1 = 1.0 / (_MAG_BINS - 1)

_NC = 2
_NS = 16
_NW = _NC * _NS
_LANES = 16
_PER_W = _N // _NW
_CHUNK = 2048
_NCHUNK = _PER_W // _CHUNK
_NPAIRS = _NCHUNK // 2


def _sc_body(cp_hbm, cm_hbm, sp_hbm, sm_hbm, cos_hbm, sin_hbm, mag_hbm,
             phase_hbm, mago_hbm, sig_hbm, part_hbm, gp_hbm, gm_hbm,
             *scratch):
    ins = (scratch[0:4], scratch[4:8])
    outs = (scratch[8:13], scratch[13:18])
    cosv, sinv, magv, accv = scratch[18:22]
    in_sems = scratch[22:24]
    out_sems = scratch[24:26]
    in_hbm = (cp_hbm, cm_hbm, sp_hbm, sm_hbm)
    out_hbm = (phase_hbm, mago_hbm, sig_hbm, gp_hbm, gm_hbm)

    wid = lax.axis_index("c") * _NS + lax.axis_index("s")
    base = wid * _PER_W

    pltpu.sync_copy(cos_hbm, cosv)
    pltpu.sync_copy(sin_hbm, sinv)
    pltpu.sync_copy(mag_hbm, magv)

    def start_in(g, b):
        sl = pl.ds(base + g * _CHUNK, _CHUNK)
        for hbm, buf in zip(in_hbm, ins[b]):
            pltpu.async_copy(hbm.at[sl], buf, in_sems[b])

    def wait_in(b):
        for hbm, buf in zip(in_hbm, ins[b]):
            pltpu.make_async_copy(hbm.at[pl.ds(0, _CHUNK)], buf,
                                  in_sems[b]).wait()

    def start_out(g, b):
        sl = pl.ds(base + g * _CHUNK, _CHUNK)
        for hbm, buf in zip(out_hbm, outs[b]):
            pltpu.async_copy(buf, hbm.at[sl], out_sems[b])

    def wait_out(b):
        for hbm, buf in zip(out_hbm, outs[b]):
            pltpu.make_async_copy(buf, hbm.at[pl.ds(0, _CHUNK)],
                                  out_sems[b]).wait()

    def compute(b, acc):
        cpv, cmv, spv, smv = ins[b]
        pov, mov, sigv, gpv, gmv = outs[b]

        @plsc.parallel_loop(0, _CHUNK, step=_LANES, unroll=2, carry=acc)
        def vec_loop(i, acc_in):
            vs = pl.ds(i, _LANES)
            p = (cpv[vs] + spv[vs]) & (_PHASE_BINS - 1)
            mg = (cmv[vs] + smv[vs]) & (_MAG_BINS - 1)
            pov[vs] = p
            mov[vs] = mg
            c = plsc.load_gather(cosv, [p])
            s = plsc.load_gather(sinv, [p])
            m = plsc.load_gather(magv, [mg])
            sig = c * m
            sigv[vs] = sig
            gpv[vs] = (s * m) * (-_TWO_PI_OVER_P)
            gmv[vs] = sig * _INV_MM1
            return acc_in + sig

        return vec_loop

    start_in(0, 0)

    def pair_body(tt, acc):
        g0 = 2 * tt
        start_in(g0 + 1, 1)
        wait_in(0)

        @pl.when(tt > 0)
        def _():
            wait_out(0)

        acc0 = compute(0, acc)
        start_out(g0, 0)

        @pl.when(tt < _NPAIRS - 1)
        def _():
            start_in(g0 + 2, 0)

        wait_in(1)

        @pl.when(tt > 0)
        def _():
            wait_out(1)

        acc1 = compute(1, acc0)
        start_out(g0 + 1, 1)
        return acc1

    acc = lax.fori_loop(0, _NPAIRS, pair_body,
                        jnp.zeros((_LANES,), jnp.float32))
    wait_out(0)
    wait_out(1)
    accv[...] = acc
    pltpu.sync_copy(accv, part_hbm.at[wid])


_sc_call = functools.partial(
    pl.kernel,
    out_type=(
        jax.ShapeDtypeStruct((_N,), jnp.int32),
        jax.ShapeDtypeStruct((_N,), jnp.int32),
        jax.ShapeDtypeStruct((_N,), jnp.float32),
        jax.ShapeDtypeStruct((_NW, _LANES), jnp.float32),
        jax.ShapeDtypeStruct((_N,), jnp.float32),
        jax.ShapeDtypeStruct((_N,), jnp.float32),
    ),
    mesh=plsc.VectorSubcoreMesh(core_axis_name="c", subcore_axis_name="s"),
    compiler_params=pltpu.CompilerParams(needs_layout_passes=False),
    scratch_types=(
        [pltpu.VMEM((_CHUNK,), jnp.int32)] * 8
        + [pltpu.VMEM((_CHUNK,), jnp.int32)] * 2
        + [pltpu.VMEM((_CHUNK,), jnp.float32)] * 3
        + [pltpu.VMEM((_CHUNK,), jnp.int32)] * 2
        + [pltpu.VMEM((_CHUNK,), jnp.float32)] * 3
        + [pltpu.VMEM((_PHASE_BINS,), jnp.float32),
           pltpu.VMEM((_PHASE_BINS,), jnp.float32),
           pltpu.VMEM((_MAG_BINS,), jnp.float32),
           pltpu.VMEM((_LANES,), jnp.float32)]
        + [pltpu.SemaphoreType.DMA] * 4
    ),
)(_sc_body)


def kernel(ctx_phase_idx, ctx_mag_idx, self_phase_idx, self_mag_idx,
           cos_table, sin_table, mag_table):
    phase_out, mag_out, signal, parts, grad_phase, grad_mag = _sc_call(
        ctx_phase_idx, ctx_mag_idx, self_phase_idx, self_mag_idx,
        cos_table, sin_table, mag_table)
    strength = jnp.sum(parts)
    return (phase_out, mag_out, signal, strength, grad_phase, grad_mag)


# speedup vs baseline: 1.0312x; 1.0312x over previous
import functools

import jax
import jax.numpy as jnp
from jax import lax
from jax.experimental import pallas as pl
from jax.experimental.pallas import tpu as pltpu
from jax.experimental.pallas import tpu_sc as plsc

_N = 4194304
_PHASE_BINS = 64
_MAG_BINS = 1024
_TWO_PI_OVER_P = 2.0 * 3.141592653589793 / _PHASE_BINS
_INV_MM1 = 1.0 / (_MAG_BINS - 1)

_NC = 2
_NS = 16
_NW = _NC * _NS
_LANES = 16
_PER_W = _N // _NW
_CHUNK = 8192
_NCHUNK = _PER_W // _CHUNK
_NPAIRS = _NCHUNK // 2


def _sc_body(cp_hbm, cm_hbm, sp_hbm, sm_hbm, cos_hbm, sin_hbm, mag_hbm,
             phase_hbm, mago_hbm, sig_hbm, part_hbm, gp_hbm, gm_hbm,
             *scratch):
    ins = (scratch[0:4], scratch[4:8])
    fouts = (scratch[8:11], scratch[11:14])
    cosv, sinv, magv, accv = scratch[14:18]
    in_sems = scratch[18:20]
    outa_sems = scratch[20:22]
    outb_sems = scratch[22:24]
    in_hbm = (cp_hbm, cm_hbm, sp_hbm, sm_hbm)

    wid = lax.axis_index("c") * _NS + lax.axis_index("s")
    base = wid * _PER_W

    pltpu.sync_copy(cos_hbm, cosv)
    pltpu.sync_copy(sin_hbm, sinv)
    pltpu.sync_copy(mag_hbm, magv)

    def start_in(g, b):
        sl = pl.ds(base + g * _CHUNK, _CHUNK)
        for hbm, buf in zip(in_hbm, ins[b]):
            pltpu.async_copy(hbm.at[sl], buf, in_sems[b])

    def wait_in(b):
        for hbm, buf in zip(in_hbm, ins[b]):
            pltpu.make_async_copy(hbm.at[pl.ds(0, _CHUNK)], buf,
                                  in_sems[b]).wait()

    def start_out(g, b):
        sl = pl.ds(base + g * _CHUNK, _CHUNK)
        cpv, cmv, _, _ = ins[b]
        sigv, gpv, gmv = fouts[b]
        pltpu.async_copy(cpv, phase_hbm.at[sl], outa_sems[b])
        pltpu.async_copy(cmv, mago_hbm.at[sl], outa_sems[b])
        pltpu.async_copy(sigv, sig_hbm.at[sl], outb_sems[b])
        pltpu.async_copy(gpv, gp_hbm.at[sl], outb_sems[b])
        pltpu.async_copy(gmv, gm_hbm.at[sl], outb_sems[b])

    def wait_outa(b):
        cpv, cmv, _, _ = ins[b]
        pltpu.make_async_copy(cpv, phase_hbm.at[pl.ds(0, _CHUNK)],
                              outa_sems[b]).wait()
        pltpu.make_async_copy(cmv, mago_hbm.at[pl.ds(0, _CHUNK)],
                              outa_sems[b]).wait()

    def wait_outb(b):
        sigv, gpv, gmv = fouts[b]
        pltpu.make_async_copy(sigv, sig_hbm.at[pl.ds(0, _CHUNK)],
                              outb_sems[b]).wait()
        pltpu.make_async_copy(gpv, gp_hbm.at[pl.ds(0, _CHUNK)],
                              outb_sems[b]).wait()
        pltpu.make_async_copy(gmv, gm_hbm.at[pl.ds(0, _CHUNK)],
                              outb_sems[b]).wait()

    def compute(b, acc):
        cpv, cmv, spv, smv = ins[b]
        sigv, gpv, gmv = fouts[b]

        @plsc.parallel_loop(0, _CHUNK, step=_LANES, unroll=2, carry=acc)
        def vec_loop(i, acc_in):
            vs = pl.ds(i, _LANES)
            p = (cpv[vs] + spv[vs]) & (_PHASE_BINS - 1)
            mg = (cmv[vs] + smv[vs]) & (_MAG_BINS - 1)
            cpv[vs] = p
            cmv[vs] = mg
            c = plsc.load_gather(cosv, [p])
            s = plsc.load_gather(sinv, [p])
            m = plsc.load_gather(magv, [mg])
            sig = c * m
            sigv[vs] = sig
            gpv[vs] = (s * m) * (-_TWO_PI_OVER_P)
            gmv[vs] = sig * _INV_MM1
            return acc_in + sig

        return vec_loop

    start_in(0, 0)
    start_in(1, 1)

    def pair_body(tt, acc):
        g0 = 2 * tt
        wait_in(0)

        @pl.when(tt > 0)
        def _():
            wait_outb(0)

        acc0 = compute(0, acc)
        start_out(g0, 0)

        @pl.when(tt < _NPAIRS - 1)
        def _():
            wait_outa(0)
            start_in(g0 + 2, 0)

        wait_in(1)

        @pl.when(tt > 0)
        def _():
            wait_outb(1)

        acc1 = compute(1, acc0)
        start_out(g0 + 1, 1)

        @pl.when(tt < _NPAIRS - 1)
        def _():
            wait_outa(1)
            start_in(g0 + 3, 1)

        return acc1

    acc = lax.fori_loop(0, _NPAIRS, pair_body,
                        jnp.zeros((_LANES,), jnp.float32))
    wait_outa(0)
    wait_outa(1)
    wait_outb(0)
    wait_outb(1)
    accv[...] = acc
    pltpu.sync_copy(accv, part_hbm.at[wid])


_sc_call = functools.partial(
    pl.kernel,
    out_type=(
        jax.ShapeDtypeStruct((_N,), jnp.int32),
        jax.ShapeDtypeStruct((_N,), jnp.int32),
        jax.ShapeDtypeStruct((_N,), jnp.float32),
        jax.ShapeDtypeStruct((_NW, _LANES), jnp.float32),
        jax.ShapeDtypeStruct((_N,), jnp.float32),
        jax.ShapeDtypeStruct((_N,), jnp.float32),
    ),
    mesh=plsc.VectorSubcoreMesh(core_axis_name="c", subcore_axis_name="s"),
    compiler_params=pltpu.CompilerParams(needs_layout_passes=False),
    scratch_types=(
        [pltpu.VMEM((_CHUNK,), jnp.int32)] * 8
        + [pltpu.VMEM((_CHUNK,), jnp.float32)] * 6
        + [pltpu.VMEM((_PHASE_BINS,), jnp.float32),
           pltpu.VMEM((_PHASE_BINS,), jnp.float32),
           pltpu.VMEM((_MAG_BINS,), jnp.float32),
           pltpu.VMEM((_LANES,), jnp.float32)]
        + [pltpu.SemaphoreType.DMA] * 6
    ),
)(_sc_body)


def kernel(ctx_phase_idx, ctx_mag_idx, self_phase_idx, self_mag_idx,
           cos_table, sin_table, mag_table):
    phase_out, mag_out, signal, parts, grad_phase, grad_mag = _sc_call(
        ctx_phase_idx, ctx_mag_idx, self_phase_idx, self_mag_idx,
        cos_table, sin_table, mag_table)
    strength = jnp.sum(parts)
    return (phase_out, mag_out, signal, strength, grad_phase, grad_mag)


# speedup vs baseline: 1.1368x; 1.1024x over previous
import functools

import jax
import jax.numpy as jnp
from jax import lax
from jax.experimental import pallas as pl
from jax.experimental.pallas import tpu as pltpu
from jax.experimental.pallas import tpu_sc as plsc

_N = 4194304
_PHASE_BINS = 64
_MAG_BINS = 1024
_TWO_PI_OVER_P = 2.0 * 3.141592653589793 / _PHASE_BINS
_INV_MM1 = 1.0 / (_MAG_BINS - 1)

_NC = 2
_NS = 16
_NW = _NC * _NS
_LANES = 16
_PER_W = _N // _NW
_CHUNK = 4096
_NCHUNK = _PER_W // _CHUNK
_NPAIRS = _NCHUNK // 2


def _sc_body(cp_hbm, cm_hbm, sp_hbm, sm_hbm, cos_hbm, sin_hbm, mag_hbm,
             phase_hbm, mago_hbm, sig_hbm, part_hbm, gp_hbm, gm_hbm,
             *scratch):
    ins = (scratch[0:4], scratch[4:8])
    outs = (scratch[8:13], scratch[13:18])
    cosv, sinv, magv, accv = scratch[18:22]
    in_sems = scratch[22:24]
    out_sems = scratch[24:26]
    in_hbm = (cp_hbm, cm_hbm, sp_hbm, sm_hbm)
    out_hbm = (phase_hbm, mago_hbm, sig_hbm, gp_hbm, gm_hbm)

    wid = lax.axis_index("c") * _NS + lax.axis_index("s")
    base = wid * _PER_W

    pltpu.sync_copy(cos_hbm, cosv)
    pltpu.sync_copy(sin_hbm, sinv)
    pltpu.sync_copy(mag_hbm, magv)

    def start_in(g, b):
        sl = pl.ds(base + g * _CHUNK, _CHUNK)
        for hbm, buf in zip(in_hbm, ins[b]):
            pltpu.async_copy(hbm.at[sl], buf, in_sems[b])

    def wait_in(b):
        for hbm, buf in zip(in_hbm, ins[b]):
            pltpu.make_async_copy(hbm.at[pl.ds(0, _CHUNK)], buf,
                                  in_sems[b]).wait()

    def start_out(g, b):
        sl = pl.ds(base + g * _CHUNK, _CHUNK)
        for hbm, buf in zip(out_hbm, outs[b]):
            pltpu.async_copy(buf, hbm.at[sl], out_sems[b])

    def wait_out(b):
        for hbm, buf in zip(out_hbm, outs[b]):
            pltpu.make_async_copy(buf, hbm.at[pl.ds(0, _CHUNK)],
                                  out_sems[b]).wait()

    def compute(b, acc):
        cpv, cmv, spv, smv = ins[b]
        pov, mov, sigv, gpv, gmv = outs[b]

        @plsc.parallel_loop(0, _CHUNK, step=_LANES, unroll=2, carry=acc)
        def vec_loop(i, acc_in):
            vs = pl.ds(i, _LANES)
            p = (cpv[vs] + spv[vs]) & (_PHASE_BINS - 1)
            mg = (cmv[vs] + smv[vs]) & (_MAG_BINS - 1)
            pov[vs] = p
            mov[vs] = mg
            c = plsc.load_gather(cosv, [p])
            s = plsc.load_gather(sinv, [p])
            m = plsc.load_gather(magv, [mg])
            sig = c * m
            sigv[vs] = sig
            gpv[vs] = (s * m) * (-_TWO_PI_OVER_P)
            gmv[vs] = sig * _INV_MM1
            return acc_in + sig

        return vec_loop

    start_in(0, 0)

    def pair_body(tt, acc):
        g0 = 2 * tt
        start_in(g0 + 1, 1)
        wait_in(0)

        @pl.when(tt > 0)
        def _():
            wait_out(0)

        acc0 = compute(0, acc)
        start_out(g0, 0)

        @pl.when(tt < _NPAIRS - 1)
        def _():
            start_in(g0 + 2, 0)

        wait_in(1)

        @pl.when(tt > 0)
        def _():
            wait_out(1)

        acc1 = compute(1, acc0)
        start_out(g0 + 1, 1)
        return acc1

    acc = lax.fori_loop(0, _NPAIRS, pair_body,
                        jnp.zeros((_LANES,), jnp.float32))
    wait_out(0)
    wait_out(1)
    accv[...] = acc
    pltpu.sync_copy(accv, part_hbm.at[wid])


_sc_call = functools.partial(
    pl.kernel,
    out_type=(
        jax.ShapeDtypeStruct((_N,), jnp.int32),
        jax.ShapeDtypeStruct((_N,), jnp.int32),
        jax.ShapeDtypeStruct((_N,), jnp.float32),
        jax.ShapeDtypeStruct((_NW, _LANES), jnp.float32),
        jax.ShapeDtypeStruct((_N,), jnp.float32),
        jax.ShapeDtypeStruct((_N,), jnp.float32),
    ),
    mesh=plsc.VectorSubcoreMesh(core_axis_name="c", subcore_axis_name="s"),
    compiler_params=pltpu.CompilerParams(needs_layout_passes=False),
    scratch_types=(
        [pltpu.VMEM((_CHUNK,), jnp.int32)] * 8
        + [pltpu.VMEM((_CHUNK,), jnp.int32)] * 2
        + [pltpu.VMEM((_CHUNK,), jnp.float32)] * 3
        + [pltpu.VMEM((_CHUNK,), jnp.int32)] * 2
        + [pltpu.VMEM((_CHUNK,), jnp.float32)] * 3
        + [pltpu.VMEM((_PHASE_BINS,), jnp.float32),
           pltpu.VMEM((_PHASE_BINS,), jnp.float32),
           pltpu.VMEM((_MAG_BINS,), jnp.float32),
           pltpu.VMEM((_LANES,), jnp.float32)]
        + [pltpu.SemaphoreType.DMA] * 4
    ),
)(_sc_body)


def kernel(ctx_phase_idx, ctx_mag_idx, self_phase_idx, self_mag_idx,
           cos_table, sin_table, mag_table):
    phase_out, mag_out, signal, parts, grad_phase, grad_mag = _sc_call(
        ctx_phase_idx, ctx_mag_idx, self_phase_idx, self_mag_idx,
        cos_table, sin_table, mag_table)
    strength = jnp.sum(parts)
    return (phase_out, mag_out, signal, strength, grad_phase, grad_mag)
